# baseline (speedup 1.0000x reference)
import functools
import jax
import jax.numpy as jnp
from jax import lax
from jax.experimental import pallas as pl
from jax.experimental.pallas import tpu as pltpu

_STRIPES = (32, 32, 32, 24, 8)

def _tc_body(B, V, stripes, logits_hbm, x_ref, xrep_ref, out_ref, picked,
             psem, *scratch):
  n = len(stripes)
  bufs = scratch[:n]
  sems = scratch[n]
  offs = [sum(stripes[:k]) for k in range(n)]

  def stripe_copy(k):
    return pltpu.make_async_copy(
        logits_hbm.at[pl.ds(offs[k], stripes[k]), :], bufs[k], sems.at[k])

  def pick_copy(r):
    base = pl.multiple_of((x_ref[0, r] // 128) * 128, 128)
    return pltpu.make_async_copy(
        logits_hbm.at[pl.ds((r // 8) * 8, 8), pl.ds(base, 128)],
        picked.at[pl.ds(r * 8, 8), :], psem)

  for k in range(n):
    stripe_copy(k).start()
  for r in range(B):
    pick_copy(r).start()

  total = jnp.zeros((1, 1), jnp.float32)
  for k in range(n):
    stripe_copy(k).wait()
    chunk = bufs[k][...]
    m = chunk.max(axis=1, keepdims=True)
    s = jnp.exp(chunk - m).sum(axis=1, keepdims=True)
    total = total - jnp.sum(m + jnp.log(s)).reshape(1, 1)

  for r in range(B):
    pick_copy(r).wait()
  xr = xrep_ref[...]
  baser = (xr // 128) * 128
  rowi = lax.broadcasted_iota(jnp.int32, (8 * B, 128), 0)
  lane = lax.broadcasted_iota(jnp.int32, (8 * B, 128), 1)
  rowsel = (rowi % 8) == ((rowi // 8) % 8)
  psel = jnp.where(rowsel & (lane == xr - baser), picked[...], 0.0)
  out_ref[...] = total + jnp.sum(psel).reshape(1, 1)

def kernel(logits, x):
  B, V = logits.shape
  x = x.astype(jnp.int32)
  xrep = jnp.repeat(x, 8).reshape(8 * B, 1)
  out = pl.pallas_call(
      functools.partial(_tc_body, B, V, _STRIPES),
      in_specs=[
          pl.BlockSpec(memory_space=pltpu.MemorySpace.HBM),
          pl.BlockSpec(memory_space=pltpu.MemorySpace.SMEM),
          pl.BlockSpec((8 * B, 1), lambda: (0, 0)),
      ],
      out_specs=pl.BlockSpec((1, 1), lambda: (0, 0)),
      out_shape=jax.ShapeDtypeStruct((1, 1), jnp.float32),
      scratch_shapes=(
          [pltpu.VMEM((8 * B, 128), jnp.float32), pltpu.SemaphoreType.DMA]
          + [pltpu.VMEM((r, V), jnp.float32) for r in _STRIPES]
          + [pltpu.SemaphoreType.DMA((len(_STRIPES),))]
      ),
  )(logits, x.reshape(1, B), xrep)
  return out[0, 0]


# speedup vs baseline: 1.0197x; 1.0197x over previous
import functools
import jax
import jax.numpy as jnp
from jax import lax
from jax.experimental import pallas as pl
from jax.experimental.pallas import tpu as pltpu

_STRIPES = (32, 32, 32, 32)

def _tc_body(B, V, stripes, logits_hbm, x_ref, xrep_ref, out_ref, picked,
             psem, *scratch):
  n = len(stripes)
  bufs = scratch[:n]
  sems = scratch[n]
  offs = [sum(stripes[:k]) for k in range(n)]

  def stripe_copy(k):
    return pltpu.make_async_copy(
        logits_hbm.at[pl.ds(offs[k], stripes[k]), :], bufs[k], sems.at[k])

  def pick_copy(r):
    base = pl.multiple_of((x_ref[0, r] // 128) * 128, 128)
    return pltpu.make_async_copy(
        logits_hbm.at[pl.ds((r // 8) * 8, 8), pl.ds(base, 128)],
        picked.at[pl.ds(r * 8, 8), :], psem)

  for k in range(n):
    stripe_copy(k).start()
  for r in range(B):
    pick_copy(r).start()

  total = jnp.zeros((1, 1), jnp.float32)
  for k in range(n):
    stripe_copy(k).wait()
    chunk = bufs[k][...]
    m = chunk.max(axis=1, keepdims=True)
    s = jnp.exp(chunk - m).sum(axis=1, keepdims=True)
    total = total - jnp.sum(m + jnp.log(s)).reshape(1, 1)

  for r in range(B):
    pick_copy(r).wait()
  xr = xrep_ref[...]
  baser = (xr // 128) * 128
  rowi = lax.broadcasted_iota(jnp.int32, (8 * B, 128), 0)
  lane = lax.broadcasted_iota(jnp.int32, (8 * B, 128), 1)
  rowsel = (rowi % 8) == ((rowi // 8) % 8)
  psel = jnp.where(rowsel & (lane == xr - baser), picked[...], 0.0)
  out_ref[...] = total + jnp.sum(psel).reshape(1, 1)

def kernel(logits, x):
  B, V = logits.shape
  x = x.astype(jnp.int32)
  xrep = jnp.repeat(x, 8).reshape(8 * B, 1)
  out = pl.pallas_call(
      functools.partial(_tc_body, B, V, _STRIPES),
      in_specs=[
          pl.BlockSpec(memory_space=pltpu.MemorySpace.HBM),
          pl.BlockSpec(memory_space=pltpu.MemorySpace.SMEM),
          pl.BlockSpec((8 * B, 1), lambda: (0, 0)),
      ],
      out_specs=pl.BlockSpec((1, 1), lambda: (0, 0)),
      out_shape=jax.ShapeDtypeStruct((1, 1), jnp.float32),
      scratch_shapes=(
          [pltpu.VMEM((8 * B, 128), jnp.float32), pltpu.SemaphoreType.DMA]
          + [pltpu.VMEM((r, V), jnp.float32) for r in _STRIPES]
          + [pltpu.SemaphoreType.DMA((len(_STRIPES),))]
      ),
  )(logits, x.reshape(1, B), xrep)
  return out[0, 0]
